# baseline (speedup 1.0000x reference)
import functools

import jax
import jax.numpy as jnp
from jax import lax
import jax.experimental.pallas as pl
from jax.experimental.pallas import tpu as pltpu
from jax.experimental.pallas import tpu_sc as plsc

NX = 704
NY = 800
C_OUT = 32
N = 400000
NW = 32
CHUNK = 12500
CW = CHUNK * 4
G = (CHUNK + 15) // 16
OSTRIDE = 720
CAPR = 32
SLABS_PER_W = NX // NW
BINPAD = 4 * (CAPR + 4)
ACCW = NY * C_OUT

_mesh = plsc.VectorSubcoreMesh(core_axis_name="c", subcore_axis_name="s")
_params = pltpu.CompilerParams(needs_layout_passes=False)


@functools.partial(
    pl.kernel,
    out_type=(
        jax.ShapeDtypeStruct((N * 4 + BINPAD,), jnp.float32),
        jax.ShapeDtypeStruct((NW * OSTRIDE,), jnp.int32),
    ),
    mesh=_mesh,
    compiler_params=_params,
    scratch_types=[
        pltpu.VMEM((G * 64,), jnp.float32),
        pltpu.VMEM((CW,), jnp.float32),
        pltpu.VMEM((G * 16,), jnp.int32),
        pltpu.VMEM((NX,), jnp.int32),
        pltpu.VMEM((NX,), jnp.int32),
        pltpu.VMEM((OSTRIDE,), jnp.int32),
    ],
)
def _bin_kernel(pts_hbm, binned_hbm, offs_hbm,
                pts_v, buf_v, ixs_v, hist_v, cur_v, offs_v):
    w = lax.axis_index("s") * 2 + lax.axis_index("c")
    base = pl.multiple_of(w * CW, 8)
    pltpu.sync_copy(pts_hbm.at[pl.ds(base, CW)], pts_v.at[pl.ds(0, CW)])

    def zero16(i, _):
        hist_v[pl.ds(i * 16, 16)] = jnp.zeros((16,), jnp.int32)
        return 0
    lax.fori_loop(0, NX // 16, zero16, 0)

    lanes = lax.iota(jnp.int32, 16)

    zero16v = jnp.zeros((16,), jnp.int32)

    def pass1(g, _):
        pid = g * 16 + lanes
        valid = pid < CHUNK
        x = plsc.load_gather(pts_v, [pid * 4])
        ix = jnp.clip((x / 0.1).astype(jnp.int32), 0, NX - 1)
        ix = jnp.where(valid, ix, 0)
        ixs_v[pl.ds(g * 16, 16)] = ix
        cnt, last = plsc.scan_count(ix, mask=valid)
        plsc.addupdate_scatter(hist_v, [ix], cnt, mask=last)
        return 0
    lax.fori_loop(0, G, pass1, 0)

    def scan_step(i, carry):
        h = hist_v[pl.ds(i * 16, 16)]
        inc = plsc.cumsum(h) + jnp.full((16,), carry, jnp.int32)
        excl = inc - h
        cur_v[pl.ds(i * 16, 16)] = excl
        offs_v[pl.ds(i * 16, 16)] = excl
        return inc[15]
    total = lax.fori_loop(0, NX // 16, scan_step, jnp.int32(0))
    offs_v[pl.ds(NX, 16)] = jnp.where(lanes == 0, total, 0)
    pltpu.sync_copy(
        offs_v, offs_hbm.at[pl.ds(pl.multiple_of(w * OSTRIDE, 8), OSTRIDE)])

    def pass2(g, _):
        pid = g * 16 + lanes
        valid = pid < CHUNK
        ix = ixs_v[pl.ds(g * 16, 16)]
        cnt, last = plsc.scan_count(ix, mask=valid)
        cur = plsc.load_gather(cur_v, [ix])
        dest = cur + cnt - 1
        plsc.store_scatter(cur_v, [ix], cur + cnt, mask=last)
        d4 = dest * 4
        for k in range(4):
            comp = plsc.load_gather(pts_v, [pid * 4 + k])
            plsc.store_scatter(buf_v, [d4 + k], comp, mask=valid)
        return 0
    lax.fori_loop(0, G, pass2, 0)

    pltpu.sync_copy(buf_v, binned_hbm.at[pl.ds(base, CW)])


@functools.partial(
    pl.kernel,
    out_type=jax.ShapeDtypeStruct((NX * NY * C_OUT,), jnp.float32),
    mesh=_mesh,
    compiler_params=_params,
    scratch_types=[
        pltpu.VMEM((NW * OSTRIDE,), jnp.int32),
        pltpu.VMEM((224,), jnp.float32),
        pltpu.VMEM((ACCW,), jnp.float32),
        pltpu.VMEM((CAPR * 4 + 16,), jnp.float32),
    ],
)
def _pool_kernel(binned_hbm, offs_hbm, coef_hbm, out_hbm,
                 offs_v, coef_v, acc_v, stage_v):
    w = lax.axis_index("s") * 2 + lax.axis_index("c")
    pltpu.sync_copy(offs_hbm, offs_v)
    pltpu.sync_copy(coef_hbm, coef_v)

    a0l = coef_v[pl.ds(0, 16)]
    a0h = coef_v[pl.ds(16, 16)]
    a1l = coef_v[pl.ds(32, 16)]
    a1h = coef_v[pl.ds(48, 16)]
    a2l = coef_v[pl.ds(64, 16)]
    a2h = coef_v[pl.ds(80, 16)]
    a3l = coef_v[pl.ds(96, 16)]
    a3h = coef_v[pl.ds(112, 16)]
    b1l = coef_v[pl.ds(128, 16)]
    b1h = coef_v[pl.ds(144, 16)]
    b0l = coef_v[pl.ds(160, 16)]
    b0h = coef_v[pl.ds(176, 16)]
    kl = coef_v[pl.ds(192, 16)]
    kh = coef_v[pl.ds(208, 16)]

    def do_slab(j, _):
        s = w + NW * j

        zf = jnp.zeros((16,), jnp.float32)

        def zacc(r, _):
            acc_v[pl.ds(r * 32, 16)] = zf
            acc_v[pl.ds(r * 32 + 16, 16)] = zf
            return 0
        lax.fori_loop(0, NY, zacc, 0)

        sxf = jnp.full((16,), s, jnp.int32).astype(jnp.float32)
        ksl = kl + b0l * sxf
        ksh = kh + b0h * sxf

        def do_src(t, _):
            o2 = offs_v[pl.ds(t * OSTRIDE + s, 16)]
            start_l = o2[0]
            cnt = o2[1] - start_l
            gstart = t * CHUNK + start_l

            def cond(c):
                return c[0] < cnt

            def chunk_body(c):
                pos = c[0]
                p0 = gstart + pos
                al = p0 & ~1
                skip = p0 - al
                m = jnp.minimum(CAPR - skip, cnt - pos)
                src = pl.multiple_of(al * 4, 8)
                pltpu.sync_copy(binned_hbm.at[pl.ds(src, CAPR * 4)],
                                stage_v.at[pl.ds(0, CAPR * 4)])

                def point(p, _):
                    q = (skip + p) * 4
                    row = stage_v[pl.ds(q, 16)]
                    iyv = jnp.clip(((row + 40.0) / 0.1).astype(jnp.int32),
                                   0, NY - 1)
                    iy = iyv[1]
                    bx = jnp.full((16,), row[0], jnp.float32)
                    by = jnp.full((16,), row[1], jnp.float32)
                    bz = jnp.full((16,), row[2], jnp.float32)
                    bt = jnp.full((16,), row[3], jnp.float32)
                    biy = jnp.full((16,), iy, jnp.int32).astype(jnp.float32)
                    hl = ksl + bx * a0l + by * a1l + bz * a2l \
                        + bt * a3l + biy * b1l
                    hh = ksh + bx * a0h + by * a1h + bz * a2h \
                        + bt * a3h + biy * b1h
                    hl = jnp.maximum(hl, 0.0)
                    hh = jnp.maximum(hh, 0.0)
                    a = iy * 32
                    acc_v[pl.ds(a, 16)] = jnp.maximum(
                        acc_v[pl.ds(a, 16)], hl)
                    acc_v[pl.ds(a + 16, 16)] = jnp.maximum(
                        acc_v[pl.ds(a + 16, 16)], hh)
                    return 0
                lax.fori_loop(0, m, point, 0)
                return (pos + m,)

            lax.while_loop(cond, chunk_body, (jnp.int32(0),))
            return 0
        lax.fori_loop(0, NW, do_src, 0)

        pltpu.sync_copy(
            acc_v, out_hbm.at[pl.ds(pl.multiple_of(s * ACCW, 8), ACCW)])
        return 0
    lax.fori_loop(0, SLABS_PER_W, do_slab, 0)


def kernel(points, W, b):
    Wf = W.astype(jnp.float32)
    a0 = Wf[0] + Wf[3] + Wf[6]
    a1 = Wf[1] + Wf[4] + Wf[7]
    a2 = Wf[2] + Wf[5] + Wf[8]
    a3 = Wf[9]
    b0 = -0.1 * Wf[3]
    b1 = -0.1 * Wf[4]
    k = (b.astype(jnp.float32) + 40.0 * (Wf[1] + Wf[4])
         + 3.0 * (Wf[2] + Wf[5]) - 2.0 * Wf[5]
         - 0.05 * Wf[3] - 0.05 * Wf[4])
    coef = jnp.concatenate([a0, a1, a2, a3, b1, b0, k]).astype(jnp.float32)

    pts = points.astype(jnp.float32).reshape(-1)
    binned, offs = _bin_kernel(pts)
    return _pool_kernel(binned, offs, coef).reshape(NX * NY, C_OUT)


# speedup vs baseline: 1.0283x; 1.0283x over previous
import functools

import jax
import jax.numpy as jnp
from jax import lax
import jax.experimental.pallas as pl
from jax.experimental.pallas import tpu as pltpu
from jax.experimental.pallas import tpu_sc as plsc

NX = 704
NY = 800
C_OUT = 32
N = 400000
NW = 32
CHUNK = 12500
CW = CHUNK * 4
G = (CHUNK + 15) // 16
OSTRIDE = 720
CAPR = 32
SLABS_PER_W = NX // NW
BINPAD = 4 * (CAPR + 4)
ACCW = NY * C_OUT

_mesh = plsc.VectorSubcoreMesh(core_axis_name="c", subcore_axis_name="s")
_params = pltpu.CompilerParams(needs_layout_passes=False)


@functools.partial(
    pl.kernel,
    out_type=(
        jax.ShapeDtypeStruct((N * 4 + BINPAD,), jnp.float32),
        jax.ShapeDtypeStruct((NW * OSTRIDE,), jnp.int32),
    ),
    mesh=_mesh,
    compiler_params=_params,
    scratch_types=[
        pltpu.VMEM((G * 64,), jnp.float32),
        pltpu.VMEM((CW,), jnp.float32),
        pltpu.VMEM((G * 16,), jnp.int32),
        pltpu.VMEM((NX,), jnp.int32),
        pltpu.VMEM((NX,), jnp.int32),
        pltpu.VMEM((OSTRIDE,), jnp.int32),
    ],
)
def _bin_kernel(pts_hbm, binned_hbm, offs_hbm,
                pts_v, buf_v, ixs_v, hist_v, cur_v, offs_v):
    w = lax.axis_index("s") * 2 + lax.axis_index("c")
    base = pl.multiple_of(w * CW, 8)
    pltpu.sync_copy(pts_hbm.at[pl.ds(base, CW)], pts_v.at[pl.ds(0, CW)])

    def zero16(i, _):
        hist_v[pl.ds(i * 16, 16)] = jnp.zeros((16,), jnp.int32)
        return 0
    lax.fori_loop(0, NX // 16, zero16, 0)

    lanes = lax.iota(jnp.int32, 16)

    zero16v = jnp.zeros((16,), jnp.int32)

    def pass1(g, _):
        pid = g * 16 + lanes
        valid = pid < CHUNK
        x = plsc.load_gather(pts_v, [pid * 4])
        ix = jnp.clip((x / 0.1).astype(jnp.int32), 0, NX - 1)
        ix = jnp.where(valid, ix, 0)
        ixs_v[pl.ds(g * 16, 16)] = ix
        cnt, last = plsc.scan_count(ix, mask=valid)
        plsc.addupdate_scatter(hist_v, [ix], cnt, mask=last)
        return 0
    lax.fori_loop(0, G, pass1, 0)

    def scan_step(i, carry):
        h = hist_v[pl.ds(i * 16, 16)]
        inc = plsc.cumsum(h) + jnp.full((16,), carry, jnp.int32)
        excl = inc - h
        cur_v[pl.ds(i * 16, 16)] = excl
        offs_v[pl.ds(i * 16, 16)] = excl
        return inc[15]
    total = lax.fori_loop(0, NX // 16, scan_step, jnp.int32(0))
    offs_v[pl.ds(NX, 16)] = jnp.where(lanes == 0, total, 0)
    pltpu.sync_copy(
        offs_v, offs_hbm.at[pl.ds(pl.multiple_of(w * OSTRIDE, 8), OSTRIDE)])

    def pass2(g, _):
        pid = g * 16 + lanes
        valid = pid < CHUNK
        ix = ixs_v[pl.ds(g * 16, 16)]
        cnt, last = plsc.scan_count(ix, mask=valid)
        cur = plsc.load_gather(cur_v, [ix])
        dest = cur + cnt - 1
        plsc.store_scatter(cur_v, [ix], cur + cnt, mask=last)
        d4 = dest * 4
        for k in range(4):
            comp = plsc.load_gather(pts_v, [pid * 4 + k])
            plsc.store_scatter(buf_v, [d4 + k], comp, mask=valid)
        return 0
    lax.fori_loop(0, G, pass2, 0)

    pltpu.sync_copy(buf_v, binned_hbm.at[pl.ds(base, CW)])


@functools.partial(
    pl.kernel,
    out_type=jax.ShapeDtypeStruct((NX * NY, C_OUT), jnp.float32),
    mesh=_mesh,
    compiler_params=_params,
    scratch_types=[
        pltpu.VMEM((NW * OSTRIDE,), jnp.int32),
        pltpu.VMEM((224,), jnp.float32),
        pltpu.VMEM((NY, C_OUT), jnp.float32),
        pltpu.VMEM((CAPR * 4 + 16,), jnp.float32),
    ],
)
def _pool_kernel(binned_hbm, offs_hbm, coef_hbm, out_hbm,
                 offs_v, coef_v, acc_v, stage_v):
    w = lax.axis_index("s") * 2 + lax.axis_index("c")
    pltpu.sync_copy(offs_hbm, offs_v)
    pltpu.sync_copy(coef_hbm, coef_v)

    a0l = coef_v[pl.ds(0, 16)]
    a0h = coef_v[pl.ds(16, 16)]
    a1l = coef_v[pl.ds(32, 16)]
    a1h = coef_v[pl.ds(48, 16)]
    a2l = coef_v[pl.ds(64, 16)]
    a2h = coef_v[pl.ds(80, 16)]
    a3l = coef_v[pl.ds(96, 16)]
    a3h = coef_v[pl.ds(112, 16)]
    b1l = coef_v[pl.ds(128, 16)]
    b1h = coef_v[pl.ds(144, 16)]
    b0l = coef_v[pl.ds(160, 16)]
    b0h = coef_v[pl.ds(176, 16)]
    kl = coef_v[pl.ds(192, 16)]
    kh = coef_v[pl.ds(208, 16)]

    def do_slab(j, _):
        s = w + NW * j

        zf = jnp.zeros((16,), jnp.float32)

        def zacc(r, _):
            acc_v[r, pl.ds(0, 16)] = zf
            acc_v[r, pl.ds(16, 16)] = zf
            return 0
        lax.fori_loop(0, NY, zacc, 0)

        sxf = jnp.full((16,), s, jnp.int32).astype(jnp.float32)
        ksl = kl + b0l * sxf
        ksh = kh + b0h * sxf

        def do_src(t, _):
            o2 = offs_v[pl.ds(t * OSTRIDE + s, 16)]
            start_l = o2[0]
            cnt = o2[1] - start_l
            gstart = t * CHUNK + start_l

            def cond(c):
                return c[0] < cnt

            def chunk_body(c):
                pos = c[0]
                p0 = gstart + pos
                al = p0 & ~1
                skip = p0 - al
                m = jnp.minimum(CAPR - skip, cnt - pos)
                src = pl.multiple_of(al * 4, 8)
                pltpu.sync_copy(binned_hbm.at[pl.ds(src, CAPR * 4)],
                                stage_v.at[pl.ds(0, CAPR * 4)])

                def point(p, _):
                    q = (skip + p) * 4
                    row = stage_v[pl.ds(q, 16)]
                    iyv = jnp.clip(((row + 40.0) / 0.1).astype(jnp.int32),
                                   0, NY - 1)
                    iy = iyv[1]
                    bx = jnp.full((16,), row[0], jnp.float32)
                    by = jnp.full((16,), row[1], jnp.float32)
                    bz = jnp.full((16,), row[2], jnp.float32)
                    bt = jnp.full((16,), row[3], jnp.float32)
                    biy = jnp.full((16,), iy, jnp.int32).astype(jnp.float32)
                    hl = ksl + bx * a0l + by * a1l + bz * a2l \
                        + bt * a3l + biy * b1l
                    hh = ksh + bx * a0h + by * a1h + bz * a2h \
                        + bt * a3h + biy * b1h
                    hl = jnp.maximum(hl, 0.0)
                    hh = jnp.maximum(hh, 0.0)
                    acc_v[iy, pl.ds(0, 16)] = jnp.maximum(
                        acc_v[iy, pl.ds(0, 16)], hl)
                    acc_v[iy, pl.ds(16, 16)] = jnp.maximum(
                        acc_v[iy, pl.ds(16, 16)], hh)
                    return 0
                lax.fori_loop(0, m, point, 0)
                return (pos + m,)

            lax.while_loop(cond, chunk_body, (jnp.int32(0),))
            return 0
        lax.fori_loop(0, NW, do_src, 0)

        pltpu.sync_copy(
            acc_v, out_hbm.at[pl.ds(pl.multiple_of(s * NY, 8), NY), :])
        return 0
    lax.fori_loop(0, SLABS_PER_W, do_slab, 0)


def kernel(points, W, b):
    Wf = W.astype(jnp.float32)
    a0 = Wf[0] + Wf[3] + Wf[6]
    a1 = Wf[1] + Wf[4] + Wf[7]
    a2 = Wf[2] + Wf[5] + Wf[8]
    a3 = Wf[9]
    b0 = -0.1 * Wf[3]
    b1 = -0.1 * Wf[4]
    k = (b.astype(jnp.float32) + 40.0 * (Wf[1] + Wf[4])
         + 3.0 * (Wf[2] + Wf[5]) - 2.0 * Wf[5]
         - 0.05 * Wf[3] - 0.05 * Wf[4])
    coef = jnp.concatenate([a0, a1, a2, a3, b1, b0, k]).astype(jnp.float32)

    pts = points.astype(jnp.float32).reshape(-1)
    binned, offs = _bin_kernel(pts)
    return _pool_kernel(binned, offs, coef)
